# baseline (speedup 1.0000x reference)
import jax
import jax.numpy as jnp
from jax import lax
from jax.experimental import pallas as pl
from jax.experimental.pallas import tpu as pltpu
from jax.experimental.pallas import tpu_sc as plsc

L = 20
B = 1024
T = 1000
N = L * B
NC = 2
NS = 16
NW = NC * NS
TCR = 40
NT = T // TCR
NBLK = L * NT
NG = B // 16


def _body(x_hbm, out_hbm, x_v, b0_v, b1_v, s0, s1):
    bufs = (b0_v, b1_v)
    sems = (s0, s1)
    wid = lax.axis_index("s") * NC + lax.axis_index("c")

    xcopy = pltpu.async_copy(x_hbm, x_v, s0)
    zeros16 = jnp.zeros((16,), jnp.float32)

    def zero_buf(buf):
        def zbody(r, c):
            for g in range(NG):
                buf[r, pl.ds(g * 16, 16)] = zeros16
            return c

        lax.fori_loop(0, TCR, zbody, 0)

    lane = lax.iota(jnp.int32, 16)
    ones16 = jnp.ones((16,), jnp.float32)

    start = (wid * NBLK) // NW
    n = ((wid + 1) * NBLK) // NW - start

    def loc(i):
        m = start + i
        l = m // NT
        t0 = (m - l * NT) * TCR
        return l, t0

    def fill(buf, i):
        l, t0 = loc(i)

        def gbody(g, c):
            xv = x_v[l, pl.ds(g * 16, 16)]
            rel = xv - t0
            inb = (rel >= 0) & (rel < TCR)
            plsc.store_scatter(buf, [rel, g * 16 + lane], ones16, mask=inb)
            return c

        lax.fori_loop(0, NG, gbody, 0)

        @pl.when(t0 == T - TCR)
        def _():
            def cbody(g, c):
                xv = x_v[l, pl.ds(g * 16, 16)]
                valid = xv >= 0
                cnt = jnp.zeros((16,), jnp.int32)
                for j in range(L):
                    xj = x_v[j, pl.ds(g * 16, 16)]
                    hit = (xj == xv) & (j <= l)
                    cnt = cnt + hit.astype(jnp.int32)
                yv = (valid & ((cnt & 1) == 0)).astype(jnp.float32)
                cs = pl.ds(g * 16, 16)
                buf[TCR - 1, cs] = buf[TCR - 1, cs] + yv
                return c

            lax.fori_loop(0, NG, cbody, 0)

    def undo(buf, i):
        l, t0 = loc(i)

        def gbody(g, c):
            xv = x_v[l, pl.ds(g * 16, 16)]
            rel = xv - t0
            inb = (rel >= 0) & (rel < TCR)
            plsc.store_scatter(buf, [rel, g * 16 + lane], zeros16, mask=inb)
            return c

        lax.fori_loop(0, NG, gbody, 0)

        @pl.when(t0 == T - TCR)
        def _():
            def zctx(g, c):
                buf[TCR - 1, pl.ds(g * 16, 16)] = zeros16
                return c

            lax.fori_loop(0, NG, zctx, 0)

    def start_dma(buf, sem, i):
        l, t0 = loc(i)
        pltpu.async_copy(buf, out_hbm.at[l, pl.ds(t0, TCR)], sem)

    def wait_dma(buf, sem, i):
        l, t0 = loc(i)
        pltpu.make_async_copy(buf, out_hbm.at[l, pl.ds(t0, TCR)], sem).wait()

    zero_buf(bufs[0])
    xcopy.wait()
    fill(bufs[0], 0)
    start_dma(bufs[0], sems[0], 0)
    zero_buf(bufs[1])
    fill(bufs[1], 1)
    start_dma(bufs[1], sems[1], 1)

    def round_body(r, c):
        for k in range(2):
            i = 2 + 2 * (r - 1) + k
            wait_dma(bufs[k], sems[k], i - 2)
            undo(bufs[k], i - 2)
            fill(bufs[k], i)
            start_dma(bufs[k], sems[k], i)
        return c

    lax.fori_loop(1, (n - 2) // 2 + 1, round_body, 0)

    @pl.when((n & 1) == 1)
    def _():
        wait_dma(bufs[0], sems[0], n - 3)
        undo(bufs[0], n - 3)
        fill(bufs[0], n - 1)
        start_dma(bufs[0], sems[0], n - 1)

    wait_dma(bufs[0], sems[0], n - 1 - ((n - 1) & 1))
    wait_dma(bufs[1], sems[1], n - 1 - (n & 1))


_mesh = plsc.VectorSubcoreMesh(
    core_axis_name="c", subcore_axis_name="s", num_cores=NC, num_subcores=NS
)

_sc_call = pl.kernel(
    _body,
    out_type=jax.ShapeDtypeStruct((L, T, B), jnp.float32),
    mesh=_mesh,
    scratch_types=[
        pltpu.VMEM((L, B), jnp.int32),
        pltpu.VMEM((TCR, B), jnp.float32),
        pltpu.VMEM((TCR, B), jnp.float32),
        pltpu.SemaphoreType.DMA,
        pltpu.SemaphoreType.DMA,
    ],
    compiler_params=pltpu.CompilerParams(needs_layout_passes=False),
)


@jax.jit
def kernel(x):
    x32 = x.astype(jnp.int32)
    out_t = _sc_call(x32)
    return out_t.transpose(0, 2, 1)
